# baseline (speedup 1.0000x reference)
import functools

import jax
import jax.numpy as jnp
from jax import lax
from jax.experimental import pallas as pl
from jax.experimental.pallas import tpu as pltpu
from jax.experimental.pallas import tpu_sc as plsc

_B, _L, _D3 = 4096, 200, 32
_MAX_DELTA = 24.0
_NW = 32
_RPW = _B // _NW
_CB = 4
_NCH = _RPW // _CB
_CE = _CB * _L
_NBLK = _CE // 16

_LOG = [2.4139023481949033e-09, 0.9999996692324246, -0.4999887596404531,
        0.33316691901131484, -0.24865820664837204, 0.19337637104801095,
        -0.14517645900894144, 0.09470379572442107, -0.04713346509412302,
        0.01514537217553713, -0.0022880603873605214]
_SIN = [6.283185032056425, -41.341616036018294, 81.60091389037262,
        -76.62655515200353, 41.403453201067876, -12.576403012644503]
_COS = [0.9999999922898466, -19.739205553483636, 64.93917219630505,
        -85.4511650182939, 60.17622317156302, -26.00049805851174,
        6.575565934510123]


_GD = lax.GatherDimensionNumbers(
    offset_dims=(), collapsed_slice_dims=(0,), start_index_map=(0,))


def _splat(vec, idx):
    return lax.gather(vec, idx[:, None], _GD, slice_sizes=(1,),
                      mode=lax.GatherScatterMode.PROMISE_IN_BOUNDS)


def _horner(coeffs, x):
    acc = jnp.full((16,), coeffs[-1], jnp.float32)
    for c in coeffs[-2::-1]:
        acc = acc * x + c
    return acc


def _sc_body(d_hbm, coef_hbm, out_hbm, dv, coefv, outv):
    f32, i32 = jnp.float32, jnp.int32
    wid = lax.axis_index("s") * 2 + lax.axis_index("c")

    pltpu.sync_copy(coef_hbm, coefv)
    cA = [coefv[pl.ds(0 + 16 * k, 16)] for k in range(2)]
    cBc = [coefv[pl.ds(32 + 16 * k, 16)] for k in range(2)]
    cC = [coefv[pl.ds(64 + 16 * k, 16)] for k in range(2)]
    cV = [coefv[pl.ds(96 + 16 * k, 16)] for k in range(2)]
    cB2 = [coefv[pl.ds(128 + 16 * k, 16)] for k in range(2)]
    cV4 = coefv[pl.ds(160, 16)]
    cB24 = coefv[pl.ds(176, 16)]
    iota = lax.iota(i32, 16)

    def chunk_body(ch, _):
        ebase = wid * (_RPW * _L) + ch * _CE
        b0 = wid * _RPW + ch * _CB
        pltpu.sync_copy(d_hbm.at[pl.ds(ebase, _CE)], dv)

        def blk_body(blk, _):
            dvec = dv[pl.ds(blk * 16, 16)]
            d = jnp.minimum(jnp.maximum(dvec, 0.0), _MAX_DELTA)
            mins = d * 60.0
            s0 = jnp.where(mins < 5.0, 1.0, 0.0).astype(f32)
            s01 = jnp.where(mins < 60.0, 1.0, 0.0).astype(f32)
            s1 = s01 - s0
            xl = _horner(_LOG, d * (1.0 / _MAX_DELTA))
            q = mins * (1.0 / 60.0)
            t = q - q.astype(i32).astype(f32)
            w = t - 0.5
            y = w * w
            sinv = -(w * _horner(_SIN, y))
            cosv = -_horner(_COS, y)
            for i in range(16):
                ii = jnp.full((16,), i, i32)
                s0s = _splat(s0, ii)
                s1s = _splat(s1, ii)
                xs = _splat(xl, ii)
                e = blk * 16 + i
                bb = ((e >= 200).astype(i32) + (e >= 400).astype(i32)
                      + (e >= 600).astype(i32))
                ll = e - bb * 200
                outv[bb, ll, pl.ds(0, 16)] = s0s * cA[0] + s1s * cBc[0] + cC[0]
                outv[bb, ll, pl.ds(16, 16)] = s0s * cA[1] + s1s * cBc[1] + cC[1]
                outv[bb, ll, pl.ds(32, 16)] = xs * cV[0] + cB2[0]
                outv[bb, ll, pl.ds(48, 16)] = xs * cV[1] + cB2[1]
                v4 = xs * cV4 + cB24
                v4 = jnp.where(iota == 14, _splat(sinv, ii), v4)
                v4 = jnp.where(iota == 15, _splat(cosv, ii), v4)
                outv[bb, ll, pl.ds(50, 16)] = v4
            return 0

        lax.fori_loop(0, _NBLK, blk_body, 0)
        pltpu.sync_copy(outv, out_hbm.at[pl.ds(b0, _CB)])
        return 0

    lax.fori_loop(0, _NCH, chunk_body, 0)


def sc_kernel(deltas_hours, scale_table, W1, b1, W2, b2):
    f32 = jnp.float32
    dflat = deltas_hours.reshape(_B * _L)
    t0, t1, t2 = scale_table[0], scale_table[1], scale_table[2]
    v = jnp.maximum(W1[:, 0], 0.0) @ W2.T
    z2 = jnp.zeros((2,), f32)
    coef = jnp.concatenate(
        [t0 - t2, t1 - t2, t2, v, b2, v[18:32], z2, b2[18:32], z2]
    ).astype(f32)

    run = functools.partial(
        pl.kernel,
        out_type=jax.ShapeDtypeStruct((_B, _L, 66), f32),
        mesh=plsc.VectorSubcoreMesh(core_axis_name="c", subcore_axis_name="s"),
        scratch_types=[
            pltpu.VMEM((_CE,), f32),
            pltpu.VMEM((192,), f32),
            pltpu.VMEM((_CB, _L, 66), f32),
        ],
    )(_sc_body)
    return run(dflat, coef)


kernel = sc_kernel


# speedup vs baseline: 1.2807x; 1.2807x over previous
import functools

import jax
import jax.numpy as jnp
from jax import lax
from jax.experimental import pallas as pl
from jax.experimental.pallas import tpu as pltpu
from jax.experimental.pallas import tpu_sc as plsc

_B, _L, _D3 = 4096, 200, 32
_MAX_DELTA = 24.0
_NW = 32
_RPW = _B // _NW
_CB = 2
_NCH = _RPW // _CB
_CE = _CB * _L
_NBLK = _CE // 16

_LOG = [2.4139023481949033e-09, 0.9999996692324246, -0.4999887596404531,
        0.33316691901131484, -0.24865820664837204, 0.19337637104801095,
        -0.14517645900894144, 0.09470379572442107, -0.04713346509412302,
        0.01514537217553713, -0.0022880603873605214]
_SIN = [6.283185032056425, -41.341616036018294, 81.60091389037262,
        -76.62655515200353, 41.403453201067876, -12.576403012644503]
_COS = [0.9999999922898466, -19.739205553483636, 64.93917219630505,
        -85.4511650182939, 60.17622317156302, -26.00049805851174,
        6.575565934510123]


_GD = lax.GatherDimensionNumbers(
    offset_dims=(), collapsed_slice_dims=(0,), start_index_map=(0,))


def _splat(vec, idx):
    return lax.gather(vec, idx[:, None], _GD, slice_sizes=(1,),
                      mode=lax.GatherScatterMode.PROMISE_IN_BOUNDS)


def _horner(coeffs, x):
    acc = jnp.full((16,), coeffs[-1], jnp.float32)
    for c in coeffs[-2::-1]:
        acc = acc * x + c
    return acc


def _sc_body(d_hbm, coef_hbm, out_hbm,
             dv0, dv1, coefv, outv0, outv1, sem_in, sem_out):
    f32, i32 = jnp.float32, jnp.int32
    wid = lax.axis_index("s") * 2 + lax.axis_index("c")
    dvs = (dv0, dv1)
    outs = (outv0, outv1)

    pltpu.sync_copy(coef_hbm, coefv)
    cA = [coefv[pl.ds(0 + 16 * k, 16)] for k in range(2)]
    cBc = [coefv[pl.ds(32 + 16 * k, 16)] for k in range(2)]
    cC = [coefv[pl.ds(64 + 16 * k, 16)] for k in range(2)]
    cV = [coefv[pl.ds(96 + 16 * k, 16)] for k in range(2)]
    cB2 = [coefv[pl.ds(128 + 16 * k, 16)] for k in range(2)]
    cV4 = coefv[pl.ds(160, 16)]
    cB24 = coefv[pl.ds(176, 16)]
    iota = lax.iota(i32, 16)
    ebase0 = wid * (_RPW * _L)

    pltpu.async_copy(d_hbm.at[pl.ds(ebase0, _CE)], dv0, sem_in)

    def compute_chunk(dv, outv):
        def blk_body(blk, _):
            dvec = dv[pl.ds(blk * 16, 16)]
            d = jnp.minimum(jnp.maximum(dvec, 0.0), _MAX_DELTA)
            mins = d * 60.0
            s0 = jnp.where(mins < 5.0, 1.0, 0.0).astype(f32)
            s01 = jnp.where(mins < 60.0, 1.0, 0.0).astype(f32)
            s1 = s01 - s0
            xl = _horner(_LOG, d * (1.0 / _MAX_DELTA))
            q = mins * (1.0 / 60.0)
            t = q - q.astype(i32).astype(f32)
            w = t - 0.5
            y = w * w
            sinv = -(w * _horner(_SIN, y))
            cosv = -_horner(_COS, y)
            for i in range(16):
                ii = jnp.full((16,), i, i32)
                s0s = _splat(s0, ii)
                s1s = _splat(s1, ii)
                xs = _splat(xl, ii)
                e = blk * 16 + i
                bb = (e >= 200).astype(i32)
                ll = e - bb * 200
                outv[bb, ll, pl.ds(0, 16)] = s0s * cA[0] + s1s * cBc[0] + cC[0]
                outv[bb, ll, pl.ds(16, 16)] = s0s * cA[1] + s1s * cBc[1] + cC[1]
                outv[bb, ll, pl.ds(32, 16)] = xs * cV[0] + cB2[0]
                outv[bb, ll, pl.ds(48, 16)] = xs * cV[1] + cB2[1]
                v4 = xs * cV4 + cB24
                v4 = jnp.where(iota == 14, _splat(sinv, ii), v4)
                v4 = jnp.where(iota == 15, _splat(cosv, ii), v4)
                outv[bb, ll, pl.ds(50, 16)] = v4
            return 0

        lax.fori_loop(0, _NBLK, blk_body, 0)

    def pair_body(p, _):
        for b in range(2):
            ch = p * 2 + b
            pltpu.make_async_copy(
                d_hbm.at[pl.ds(0, _CE)], dvs[b], sem_in).wait()

            @pl.when(ch + 1 < _NCH)
            def _():
                pltpu.async_copy(
                    d_hbm.at[pl.ds(ebase0 + (ch + 1) * _CE, _CE)],
                    dvs[1 - b], sem_in)

            @pl.when(ch >= 2)
            def _():
                pltpu.make_async_copy(
                    outs[b], out_hbm.at[pl.ds(0, _CB)], sem_out).wait()

            compute_chunk(dvs[b], outs[b])
            b0 = wid * _RPW + ch * _CB
            pltpu.async_copy(outs[b], out_hbm.at[pl.ds(b0, _CB)], sem_out)
        return 0

    lax.fori_loop(0, _NCH // 2, pair_body, 0)
    pltpu.make_async_copy(outs[0], out_hbm.at[pl.ds(0, _CB)], sem_out).wait()
    pltpu.make_async_copy(outs[1], out_hbm.at[pl.ds(0, _CB)], sem_out).wait()


def sc_kernel(deltas_hours, scale_table, W1, b1, W2, b2):
    f32 = jnp.float32
    dflat = deltas_hours.reshape(_B * _L)
    t0, t1, t2 = scale_table[0], scale_table[1], scale_table[2]
    v = jnp.maximum(W1[:, 0], 0.0) @ W2.T
    z2 = jnp.zeros((2,), f32)
    coef = jnp.concatenate(
        [t0 - t2, t1 - t2, t2, v, b2, v[18:32], z2, b2[18:32], z2]
    ).astype(f32)

    run = functools.partial(
        pl.kernel,
        out_type=jax.ShapeDtypeStruct((_B, _L, 66), f32),
        mesh=plsc.VectorSubcoreMesh(core_axis_name="c", subcore_axis_name="s"),
        scratch_types=[
            pltpu.VMEM((_CE,), f32),
            pltpu.VMEM((_CE,), f32),
            pltpu.VMEM((192,), f32),
            pltpu.VMEM((_CB, _L, 66), f32),
            pltpu.VMEM((_CB, _L, 66), f32),
            pltpu.SemaphoreType.DMA,
            pltpu.SemaphoreType.DMA,
        ],
    )(_sc_body)
    return run(dflat, coef)


kernel = sc_kernel
